# Initial kernel scaffold; baseline (speedup 1.0000x reference)
import jax
import jax.numpy as jnp
from jax.experimental import pallas as pl


def kernel(x, Wg, Wu, Wd, Wr, Ag, Bg, Au, Bu, Ad, Bd):
    raise NotImplementedError("write your pallas kernel here")


# speedup vs baseline: 2.7386x; 2.7386x over previous
import functools

import jax
import jax.numpy as jnp
from jax.experimental import pallas as pl
from jax.experimental.pallas import tpu as pltpu

D = 1024
FF = 2816
E = 64
TOPK = 8
R = 16
SCALING = 2.0

FT = 256
NFT = FF // FT
T = 128
NT = 192
P = NT * T


def _k1_body(x_ref, wg_ref, wu_ref, wd_ref, wr_ref,
             gate_ref, up_ref, out_ref, logits_ref):
    f = pl.program_id(0)
    x = x_ref[...]
    g = jax.lax.dot_general(x, wg_ref[...], (((1,), (1,)), ((), ())),
                            preferred_element_type=jnp.float32)
    u = jax.lax.dot_general(x, wu_ref[...], (((1,), (1,)), ((), ())),
                            preferred_element_type=jnp.float32)
    gate_ref[...] = g
    up_ref[...] = u
    h = (g / (1.0 + jnp.exp(-g))) * u
    part = jax.lax.dot_general(h, wd_ref[...], (((1,), (1,)), ((), ())),
                               preferred_element_type=jnp.float32)

    @pl.when(f == 0)
    def _():
        out_ref[...] = part
        logits_ref[...] = jax.lax.dot_general(
            x, wr_ref[...], (((1,), (1,)), ((), ())),
            preferred_element_type=jnp.float32)

    @pl.when(f != 0)
    def _():
        out_ref[...] += part


def _base_mlp(xf, Wg, Wu, Wd, Wr):
    S = xf.shape[0]
    return pl.pallas_call(
        _k1_body,
        grid=(NFT,),
        in_specs=[
            pl.BlockSpec((S, D), lambda f: (0, 0)),
            pl.BlockSpec((FT, D), lambda f: (f, 0)),
            pl.BlockSpec((FT, D), lambda f: (f, 0)),
            pl.BlockSpec((D, FT), lambda f: (0, f)),
            pl.BlockSpec((E, D), lambda f: (0, 0)),
        ],
        out_specs=[
            pl.BlockSpec((S, FT), lambda f: (0, f)),
            pl.BlockSpec((S, FT), lambda f: (0, f)),
            pl.BlockSpec((S, D), lambda f: (0, 0)),
            pl.BlockSpec((S, E), lambda f: (0, 0)),
        ],
        out_shape=[
            jax.ShapeDtypeStruct((S, FF), jnp.float32),
            jax.ShapeDtypeStruct((S, FF), jnp.float32),
            jax.ShapeDtypeStruct((S, D), jnp.float32),
            jax.ShapeDtypeStruct((S, E), jnp.float32),
        ],
    )(xf, Wg, Wu, Wd, Wr)


def _k3_body(te_ref, gs_ref, us_ref, xs_ref, w_ref,
             ag_ref, bgt_ref, au_ref, but_ref, ad_ref, bdt_ref,
             delta_ref):
    xs = xs_ref[...]
    xag = jax.lax.dot_general(xs, ag_ref[...], (((1,), (1,)), ((), ())),
                              preferred_element_type=jnp.float32)
    gd = jax.lax.dot_general(xag, bgt_ref[...], (((1,), (0,)), ((), ())),
                             preferred_element_type=jnp.float32)
    xau = jax.lax.dot_general(xs, au_ref[...], (((1,), (1,)), ((), ())),
                              preferred_element_type=jnp.float32)
    ud = jax.lax.dot_general(xau, but_ref[...], (((1,), (0,)), ((), ())),
                             preferred_element_type=jnp.float32)
    gate = gs_ref[...] + SCALING * gd
    up = us_ref[...] + SCALING * ud
    hidden = (gate / (1.0 + jnp.exp(-gate))) * up
    had = jax.lax.dot_general(hidden, ad_ref[...], (((1,), (1,)), ((), ())),
                              preferred_element_type=jnp.float32)
    had = had * (SCALING * w_ref[...])
    delta_ref[...] = jax.lax.dot_general(
        had, bdt_ref[...], (((1,), (0,)), ((), ())),
        preferred_element_type=jnp.float32)


def _expert_deltas(tile_expert, gates_s, ups_s, xs_s, w_col,
                   Ag, BgT, Au, BuT, Ad, BdT):
    grid_spec = pltpu.PrefetchScalarGridSpec(
        num_scalar_prefetch=1,
        grid=(NT,),
        in_specs=[
            pl.BlockSpec((T, FF), lambda j, te: (j, 0)),
            pl.BlockSpec((T, FF), lambda j, te: (j, 0)),
            pl.BlockSpec((T, D), lambda j, te: (j, 0)),
            pl.BlockSpec((T, 1), lambda j, te: (j, 0)),
            pl.BlockSpec((None, R, D), lambda j, te: (te[j], 0, 0)),
            pl.BlockSpec((None, R, FF), lambda j, te: (te[j], 0, 0)),
            pl.BlockSpec((None, R, D), lambda j, te: (te[j], 0, 0)),
            pl.BlockSpec((None, R, FF), lambda j, te: (te[j], 0, 0)),
            pl.BlockSpec((None, R, FF), lambda j, te: (te[j], 0, 0)),
            pl.BlockSpec((None, R, D), lambda j, te: (te[j], 0, 0)),
        ],
        out_specs=pl.BlockSpec((T, D), lambda j, te: (j, 0)),
    )
    return pl.pallas_call(
        _k3_body,
        grid_spec=grid_spec,
        out_shape=jax.ShapeDtypeStruct((P, D), jnp.float32),
    )(tile_expert, gates_s, ups_s, xs_s, w_col,
      Ag, BgT, Au, BuT, Ad, BdT)


def _dispatch(sel, rw):
    npairs = sel.size
    expert_flat = sel.reshape(-1).astype(jnp.int32)
    sort_idx = jnp.argsort(expert_flat, stable=True).astype(jnp.int32)
    sorted_e = expert_flat[sort_idx]
    sorted_tok = (sort_idx // TOPK).astype(jnp.int32)
    sorted_w = rw.reshape(-1)[sort_idx]
    counts = jnp.bincount(expert_flat, length=E)
    padded = ((counts + T - 1) // T) * T
    pend = jnp.cumsum(padded)
    poff = pend - padded
    coff = jnp.cumsum(counts) - counts
    rank = jnp.arange(npairs, dtype=jnp.int32) - coff[sorted_e].astype(jnp.int32)
    ppos = (poff[sorted_e].astype(jnp.int32) + rank).astype(jnp.int32)
    tile_expert = jnp.searchsorted(
        pend, jnp.arange(NT, dtype=jnp.int32) * T, side='right')
    tile_expert = jnp.minimum(tile_expert, E - 1).astype(jnp.int32)
    pair_token = jnp.zeros((P,), jnp.int32).at[ppos].set(sorted_tok)
    pair_w = jnp.zeros((P,), jnp.float32).at[ppos].set(sorted_w)
    inv_pos = jnp.zeros((npairs,), jnp.int32).at[sort_idx].set(ppos)
    return tile_expert, pair_token, pair_w, inv_pos


def kernel(x, Wg, Wu, Wd, Wr, Ag, Bg, Au, Bu, Ad, Bd):
    b, s, d = x.shape
    xf = x.reshape(-1, d)

    gate_base, up_base, base_out, logits = _base_mlp(xf, Wg, Wu, Wd, Wr)

    probs = jax.nn.softmax(logits, axis=-1)
    rw, sel = jax.lax.top_k(probs, TOPK)

    tile_expert, pair_token, pair_w, inv_pos = _dispatch(sel, rw)

    gates_s = gate_base[pair_token]
    ups_s = up_base[pair_token]
    xs_s = xf[pair_token]

    BgT = jnp.swapaxes(Bg, 1, 2)
    BuT = jnp.swapaxes(Bu, 1, 2)
    BdT = jnp.swapaxes(Bd, 1, 2)

    delta = _expert_deltas(tile_expert, gates_s, ups_s, xs_s,
                           pair_w.reshape(P, 1), Ag, BgT, Au, BuT, Ad, BdT)

    expert_out = delta[inv_pos].reshape(b * s, TOPK, d).sum(axis=1)
    return (base_out + expert_out).reshape(b, s, d)
